# baseline (speedup 1.0000x reference)
import functools

import jax
import jax.numpy as jnp
from jax import lax
from jax.experimental import pallas as pl
from jax.experimental.pallas import tpu as pltpu
from jax.experimental.pallas import tpu_sc as plsc

_NC = 2
_NS = 16
_NW = _NC * _NS
_L = 16

_CH = 1
_NBUF = 4
_PAD = 8

_BS = 2048
_RT = 64


def _sc_part(x, target, tm, bs):
    B, C = x.shape
    b_per_w = bs // _NW
    n_chunks = b_per_w // _CH
    n_outer = n_chunks // _NBUF
    wvec = C // _L

    tgt_pad = jnp.zeros((bs // _CH, _PAD), jnp.int32)
    tgt_pad = tgt_pad.at[:, :_CH].set(target[:bs].reshape(bs // _CH, _CH))
    tgt_pad = tgt_pad.reshape(bs // _CH * _PAD)

    mesh = plsc.VectorSubcoreMesh(
        core_axis_name="c", subcore_axis_name="s",
        num_cores=_NC, num_subcores=_NS)

    @functools.partial(
        pl.kernel,
        out_type=(
            jax.ShapeDtypeStruct((bs, _L), jnp.float32),
            jax.ShapeDtypeStruct((bs, _L), jnp.float32),
            jax.ShapeDtypeStruct((bs, _L), jnp.float32),
            jax.ShapeDtypeStruct((bs, _L), jnp.float32),
        ),
        mesh=mesh,
        scratch_types=[
            pltpu.VMEM((n_chunks * _PAD,), jnp.int32),
            pltpu.VMEM((_NBUF, _CH, C), jnp.float32),
            pltpu.VMEM((_NBUF, _CH, C), jnp.float32),
            pltpu.VMEM((b_per_w, _L), jnp.float32),
            pltpu.VMEM((b_per_w, _L), jnp.float32),
            pltpu.VMEM((b_per_w, _L), jnp.float32),
            pltpu.VMEM((b_per_w, _L), jnp.float32),
        ] + [pltpu.SemaphoreType.DMA] * (2 * _NBUF),
    )
    def sc_kernel(x_hbm, tgt_hbm, tm_hbm, dotp_hbm, sp_hbm, mx_hbm, sep_hbm,
                  idx_v, tm_buf, x_buf, dotp_v, sp_v, mx_v, sep_v,
                  *sems):
        sems_tm = sems[:_NBUF]
        sems_x = sems[_NBUF:]
        wid = lax.axis_index("s") * _NC + lax.axis_index("c")
        base = pl.multiple_of(wid * b_per_w, b_per_w)

        chunk0 = base // _CH
        pltpu.sync_copy(
            tgt_hbm.at[pl.ds(chunk0 * _PAD, n_chunks * _PAD)], idx_v)

        def tm_copy(g, b):
            off = pl.multiple_of(g * _PAD, _PAD)
            return pltpu.make_async_copy(
                tm_hbm.at[idx_v.at[pl.ds(off, _CH)]],
                tm_buf.at[b], sems_tm[b])

        def x_copy(g, b):
            return pltpu.make_async_copy(
                x_hbm.at[pl.ds(base + g * _CH, _CH)],
                x_buf.at[b], sems_x[b])

        def start(g, b):
            tm_copy(g, b).start()
            x_copy(g, b).start()

        for b in range(_NBUF):
            start(b, b)

        def compute_chunk(g, b):
            for r in range(_CH):
                def mbody(j, m16, b=b, r=r):
                    return jnp.maximum(m16, x_buf[b, r, pl.ds(j * _L, _L)])

                m16 = lax.fori_loop(
                    0, wvec, mbody,
                    jnp.full((_L,), -jnp.inf, jnp.float32), unroll=16)

                def jbody(j, carry, b=b, r=r, m16=m16):
                    a, s, se = carry
                    off = j * _L
                    tv = tm_buf[b, r, pl.ds(off, _L)]
                    xv = x_buf[b, r, pl.ds(off, _L)]
                    return a + tv * xv, s + tv, se + jnp.exp(xv - m16)

                z = jnp.zeros((_L,), jnp.float32)
                acc, sa, se = lax.fori_loop(
                    0, wvec, jbody, (z, z, z), unroll=16)
                row = g * _CH + r
                dotp_v[row] = acc
                sp_v[row] = sa
                mx_v[row] = m16
                sep_v[row] = se

        def outer(gg, _):
            for b in range(_NBUF):
                g = gg * _NBUF + b
                tm_copy(g, b).wait()
                x_copy(g, b).wait()
                compute_chunk(g, b)
                nxt = g + _NBUF

                @pl.when(nxt < n_chunks)
                def _():
                    start(nxt, b)
            return _

        lax.fori_loop(0, n_outer, outer, None)

        pltpu.sync_copy(dotp_v, dotp_hbm.at[pl.ds(base, b_per_w)])
        pltpu.sync_copy(sp_v, sp_hbm.at[pl.ds(base, b_per_w)])
        pltpu.sync_copy(mx_v, mx_hbm.at[pl.ds(base, b_per_w)])
        pltpu.sync_copy(sep_v, sep_hbm.at[pl.ds(base, b_per_w)])

    return sc_kernel(x, tgt_pad, tm)


def _tc_part(x, target, tm, bs):
    B, C = x.shape
    n_steps = (B - bs) // _RT
    tgt_tc = target[bs:]

    def body(tgt_ref, x_ref, tm_ref, o_ref, gbuf, sems):
        i = pl.program_id(0)
        n = pl.num_programs(0)

        def gather(step, slot):
            cps = []
            for k in range(_RT):
                t = tgt_ref[step * _RT + k]
                cps.append(pltpu.make_async_copy(
                    tm_ref.at[pl.ds(t, 1)],
                    gbuf.at[slot, pl.ds(k, 1)],
                    sems.at[slot]))
            return cps

        @pl.when(i == 0)
        def _():
            for cp in gather(0, 0):
                cp.start()

        @pl.when(i + 1 < n)
        def _():
            for cp in gather(i + 1, (i + 1) % 2):
                cp.start()

        for cp in gather(i, i % 2):
            cp.wait()

        xb = x_ref[...]
        m = jnp.max(xb, axis=1, keepdims=True)
        se = jnp.sum(jnp.exp(xb - m), axis=1)
        lse = m[:, 0] + jnp.log(se)
        tmb = gbuf[i % 2]
        ss = jnp.sum(tmb, axis=1)
        part = jnp.sum(lse * ss) - jnp.sum(tmb * xb)

        @pl.when(i == 0)
        def _():
            o_ref[...] = jnp.zeros_like(o_ref)

        o_ref[...] += part.reshape(1, 1)

    grid_spec = pltpu.PrefetchScalarGridSpec(
        num_scalar_prefetch=1,
        grid=(n_steps,),
        in_specs=[
            pl.BlockSpec((_RT, C), lambda i, tgt: (i + bs // _RT, 0)),
            pl.BlockSpec(memory_space=pl.ANY),
        ],
        out_specs=pl.BlockSpec((1, 1), lambda i, tgt: (0, 0)),
        scratch_shapes=[
            pltpu.VMEM((2, _RT, C), jnp.float32),
            pltpu.SemaphoreType.DMA((2,)),
        ],
    )
    return pl.pallas_call(
        body,
        grid_spec=grid_spec,
        out_shape=jax.ShapeDtypeStruct((1, 1), jnp.float32),
    )(tgt_tc, x, tm)


def _combine(dotp, sp, mx, sep, tc_part, B):
    def body(dp_ref, sp_ref, mx_ref, sep_ref, tc_ref, o_ref):
        s = jnp.sum(sp_ref[...], axis=1)
        dot = jnp.sum(dp_ref[...], axis=1)
        mx = mx_ref[...]
        m = jnp.max(mx, axis=1, keepdims=True)
        se = jnp.sum(jnp.exp(mx - m) * sep_ref[...], axis=1)
        lse = m[:, 0] + jnp.log(se)
        tot = jnp.sum(lse * s - dot) + tc_ref[0, 0]
        o_ref[...] = (tot / B).reshape(1, 1)

    out = pl.pallas_call(
        body,
        out_shape=jax.ShapeDtypeStruct((1, 1), jnp.float32),
    )(dotp, sp, mx, sep, tc_part)
    return out[0, 0]


def kernel(x, target, target_matrix):
    B = x.shape[0]
    dotp, sp, mx, sep = _sc_part(x, target, target_matrix, _BS)
    tc_part = _tc_part(x, target, target_matrix, _BS)
    return _combine(dotp, sp, mx, sep, tc_part, B)


# speedup vs baseline: 1.0041x; 1.0041x over previous
import functools

import jax
import jax.numpy as jnp
from jax import lax
from jax.experimental import pallas as pl
from jax.experimental.pallas import tpu as pltpu
from jax.experimental.pallas import tpu_sc as plsc

_NC = 2
_NS = 16
_NW = _NC * _NS
_L = 16

_CH = 1
_NBUF = 4
_PAD = 8

_BS = 2048
_RT = 64


def _sc_part(x, target, tm, bs):
    B, C = x.shape
    b_per_w = bs // _NW
    n_chunks = b_per_w // _CH
    n_outer = n_chunks // _NBUF
    wvec = C // _L

    tgt_pad = jnp.zeros((bs // _CH, _PAD), jnp.int32)
    tgt_pad = tgt_pad.at[:, :_CH].set(target[:bs].reshape(bs // _CH, _CH))
    tgt_pad = tgt_pad.reshape(bs // _CH * _PAD)

    mesh = plsc.VectorSubcoreMesh(
        core_axis_name="c", subcore_axis_name="s",
        num_cores=_NC, num_subcores=_NS)

    @functools.partial(
        pl.kernel,
        out_type=(
            jax.ShapeDtypeStruct((bs, _L), jnp.float32),
            jax.ShapeDtypeStruct((bs, _L), jnp.float32),
            jax.ShapeDtypeStruct((bs, _L), jnp.float32),
            jax.ShapeDtypeStruct((bs, _L), jnp.float32),
        ),
        mesh=mesh,
        scratch_types=[
            pltpu.VMEM((n_chunks * _PAD,), jnp.int32),
            pltpu.VMEM((_NBUF, _CH, C), jnp.float32),
            pltpu.VMEM((_NBUF, _CH, C), jnp.float32),
            pltpu.VMEM((b_per_w, _L), jnp.float32),
            pltpu.VMEM((b_per_w, _L), jnp.float32),
            pltpu.VMEM((b_per_w, _L), jnp.float32),
            pltpu.VMEM((b_per_w, _L), jnp.float32),
        ] + [pltpu.SemaphoreType.DMA] * (2 * _NBUF),
    )
    def sc_kernel(x_hbm, tgt_hbm, tm_hbm, dotp_hbm, sp_hbm, mx_hbm, sep_hbm,
                  idx_v, tm_buf, x_buf, dotp_v, sp_v, mx_v, sep_v,
                  *sems):
        sems_tm = sems[:_NBUF]
        sems_x = sems[_NBUF:]
        wid = lax.axis_index("s") * _NC + lax.axis_index("c")
        base = pl.multiple_of(wid * b_per_w, b_per_w)

        chunk0 = base // _CH
        pltpu.sync_copy(
            tgt_hbm.at[pl.ds(chunk0 * _PAD, n_chunks * _PAD)], idx_v)

        def tm_copy(g, b):
            off = pl.multiple_of(g * _PAD, _PAD)
            return pltpu.make_async_copy(
                tm_hbm.at[idx_v.at[pl.ds(off, _CH)]],
                tm_buf.at[b], sems_tm[b])

        def x_copy(g, b):
            return pltpu.make_async_copy(
                x_hbm.at[pl.ds(base + g * _CH, _CH)],
                x_buf.at[b], sems_x[b])

        def start(g, b):
            tm_copy(g, b).start()
            x_copy(g, b).start()

        for b in range(_NBUF):
            start(b, b)

        S = 8

        def compute_chunk(g, b):
            for r in range(_CH):
                def jbody(j, carry, b=b, r=r):
                    m16, se, acc, sa = carry
                    xs, tvs = [], []
                    for s in range(S):
                        off = (j * S + s) * _L
                        xs.append(x_buf[b, r, pl.ds(off, _L)])
                        tvs.append(tm_buf[b, r, pl.ds(off, _L)])
                    mloc = xs[0]
                    for s in range(1, S):
                        mloc = jnp.maximum(mloc, xs[s])
                    m_new = jnp.maximum(m16, mloc)
                    se = se * jnp.exp(m16 - m_new)
                    for s in range(S):
                        se = se + jnp.exp(xs[s] - m_new)
                        acc = acc + tvs[s] * xs[s]
                        sa = sa + tvs[s]
                    return m_new, se, acc, sa

                z = jnp.zeros((_L,), jnp.float32)
                m16, se, acc, sa = lax.fori_loop(
                    0, wvec // S, jbody,
                    (jnp.full((_L,), -jnp.inf, jnp.float32), z, z, z))
                row = g * _CH + r
                dotp_v[row] = acc
                sp_v[row] = sa
                mx_v[row] = m16
                sep_v[row] = se

        def outer(gg, _):
            for b in range(_NBUF):
                g = gg * _NBUF + b
                tm_copy(g, b).wait()
                x_copy(g, b).wait()
                compute_chunk(g, b)
                nxt = g + _NBUF

                @pl.when(nxt < n_chunks)
                def _():
                    start(nxt, b)
            return _

        lax.fori_loop(0, n_outer, outer, None)

        pltpu.sync_copy(dotp_v, dotp_hbm.at[pl.ds(base, b_per_w)])
        pltpu.sync_copy(sp_v, sp_hbm.at[pl.ds(base, b_per_w)])
        pltpu.sync_copy(mx_v, mx_hbm.at[pl.ds(base, b_per_w)])
        pltpu.sync_copy(sep_v, sep_hbm.at[pl.ds(base, b_per_w)])

    return sc_kernel(x, tgt_pad, tm)


def _tc_part(x, target, tm, bs):
    B, C = x.shape
    n_steps = (B - bs) // _RT
    tgt_tc = target[bs:]

    def body(tgt_ref, x_ref, tm_ref, o_ref, gbuf, sems):
        i = pl.program_id(0)
        n = pl.num_programs(0)

        def gather(step, slot):
            cps = []
            for k in range(_RT):
                t = tgt_ref[step * _RT + k]
                cps.append(pltpu.make_async_copy(
                    tm_ref.at[pl.ds(t, 1)],
                    gbuf.at[slot, pl.ds(k, 1)],
                    sems.at[slot]))
            return cps

        @pl.when(i == 0)
        def _():
            for cp in gather(0, 0):
                cp.start()

        @pl.when(i + 1 < n)
        def _():
            for cp in gather(i + 1, (i + 1) % 2):
                cp.start()

        for cp in gather(i, i % 2):
            cp.wait()

        xb = x_ref[...]
        m = jnp.max(xb, axis=1, keepdims=True)
        se = jnp.sum(jnp.exp(xb - m), axis=1)
        lse = m[:, 0] + jnp.log(se)
        tmb = gbuf[i % 2]
        ss = jnp.sum(tmb, axis=1)
        part = jnp.sum(lse * ss) - jnp.sum(tmb * xb)

        @pl.when(i == 0)
        def _():
            o_ref[...] = jnp.zeros_like(o_ref)

        o_ref[...] += part.reshape(1, 1)

    grid_spec = pltpu.PrefetchScalarGridSpec(
        num_scalar_prefetch=1,
        grid=(n_steps,),
        in_specs=[
            pl.BlockSpec((_RT, C), lambda i, tgt: (i + bs // _RT, 0)),
            pl.BlockSpec(memory_space=pl.ANY),
        ],
        out_specs=pl.BlockSpec((1, 1), lambda i, tgt: (0, 0)),
        scratch_shapes=[
            pltpu.VMEM((2, _RT, C), jnp.float32),
            pltpu.SemaphoreType.DMA((2,)),
        ],
    )
    return pl.pallas_call(
        body,
        grid_spec=grid_spec,
        out_shape=jax.ShapeDtypeStruct((1, 1), jnp.float32),
    )(tgt_tc, x, tm)


def _combine(dotp, sp, mx, sep, tc_part, B):
    def body(dp_ref, sp_ref, mx_ref, sep_ref, tc_ref, o_ref):
        s = jnp.sum(sp_ref[...], axis=1)
        dot = jnp.sum(dp_ref[...], axis=1)
        mx = mx_ref[...]
        m = jnp.max(mx, axis=1, keepdims=True)
        se = jnp.sum(jnp.exp(mx - m) * sep_ref[...], axis=1)
        lse = m[:, 0] + jnp.log(se)
        tot = jnp.sum(lse * s - dot) + tc_ref[0, 0]
        o_ref[...] = (tot / B).reshape(1, 1)

    out = pl.pallas_call(
        body,
        out_shape=jax.ShapeDtypeStruct((1, 1), jnp.float32),
    )(dotp, sp, mx, sep, tc_part)
    return out[0, 0]


def kernel(x, target, target_matrix):
    B = x.shape[0]
    dotp, sp, mx, sep = _sc_part(x, target, target_matrix, _BS)
    tc_part = _tc_part(x, target, target_matrix, _BS)
    return _combine(dotp, sp, mx, sep, tc_part, B)
